# baseline (speedup 1.0000x reference)
import functools
import math

import jax
import jax.numpy as jnp
from jax.experimental import pallas as pl
from jax.experimental.pallas import tpu as pltpu

DELTA = 1e-5
TOPK_N = 128


def _soft_threshold_block(x_ref, out_ref, cur_ref, *, blk, n):
    xb = x_ref[...]
    m = jnp.max(xb, axis=1, keepdims=True)
    d = xb - m
    out_ref[...] = jnp.maximum(d, 0.0)
    v2 = jnp.max(jnp.where(d < 0.0, d, -1.0))

    @pl.when(v2 >= -DELTA)
    def _heavy():
        cur_ref[...] = jnp.maximum(d, -DELTA)
        lane = jax.lax.broadcasted_iota(jnp.int32, (blk, TOPK_N), 1)

        def body(_, carry):
            acc, filled = carry
            cur = cur_ref[...]
            v = jnp.max(cur, axis=1, keepdims=True)
            eqm = cur == v
            q = jnp.sum(eqm.astype(jnp.int32), axis=1, keepdims=True)
            cur_ref[...] = jnp.where(eqm, -3.0 * DELTA, cur)
            emit = (lane >= filled) & (lane < filled + q)
            acc = jnp.where(emit, v, acc)
            return acc, filled + q

        acc0 = jnp.zeros((blk, TOPK_N), jnp.float32)
        fill0 = jnp.zeros((blk, 1), jnp.int32)
        acc, _ = jax.lax.fori_loop(0, TOPK_N, body, (acc0, fill0))

        tri = (jax.lax.broadcasted_iota(jnp.int32, (TOPK_N, TOPK_N), 0)
               <= jax.lax.broadcasted_iota(jnp.int32, (TOPK_N, TOPK_N), 1)
               ).astype(jnp.float32)
        c = jax.lax.dot_general(acc, tri, (((1,), (0,)), ((), ())),
                                preferred_element_type=jnp.float32)
        ranks = (lane + 1).astype(jnp.float32)
        ratio = c / (acc + 1e-8)
        maskk = ranks <= ratio
        supp = jnp.clip(jnp.sum(maskk.astype(jnp.int32), axis=1,
                                keepdims=True), 1, None)
        csel = jnp.sum(jnp.where(lane == supp - 1, c, 0.0), axis=1,
                       keepdims=True)
        tau = csel / (supp.astype(jnp.float32) + 1e-8)
        out_ref[...] = jnp.maximum(d - tau, 0.0)


@jax.jit
def kernel(x):
    b, h, n = x.shape
    rows = b * h
    blk = math.gcd(64, rows)
    xf = x.reshape(rows, n)
    body = functools.partial(_soft_threshold_block, blk=blk, n=n)
    out = pl.pallas_call(
        body,
        grid=(rows // blk,),
        in_specs=[pl.BlockSpec((blk, n), lambda i: (i, 0))],
        out_specs=pl.BlockSpec((blk, n), lambda i: (i, 0)),
        out_shape=jax.ShapeDtypeStruct((rows, n), jnp.float32),
        scratch_shapes=[pltpu.VMEM((blk, n), jnp.float32)],
        compiler_params=pltpu.CompilerParams(
            dimension_semantics=("parallel",)),
    )(xf)
    return out.reshape(b, h, n)
